# baseline (speedup 1.0000x reference)
import functools

import jax
import jax.numpy as jnp
from jax import lax
from jax.experimental import pallas as pl
from jax.experimental.pallas import tpu as pltpu
from jax.experimental.pallas import tpu_sc as plsc

_NUM_CORES = 2
_NUM_SUBCORES = 16
_NUM_WORKERS = _NUM_CORES * _NUM_SUBCORES
_LANES = 16

_LA = 104

_NBUF = 4

_PERM = sum(
    ([32 * g + 2 * k for k in range(16)] + [32 * g + 2 * k + 1 for k in range(16)]
     for g in range(2)),
    [],
)


_H = 19200
_G = 26
_MAIN = 2 * _H * _G
_TAIL_OUT = 800


@functools.lru_cache(maxsize=None)
def _make_reformat(V, D):
    H = _H
    assert _MAIN + 2 * _TAIL_OUT == V and (V // 2) % _TAIL_OUT == 0

    def body(in0_ref, in1_ref, o_ref):
        m = jnp.concatenate([in0_ref[...], in1_ref[...]], axis=0)
        o_ref[...] = jnp.transpose(m)

    main = pl.pallas_call(
        body,
        grid=(_G,),
        in_specs=[
            pl.BlockSpec((D, H), lambda i: (0, 2 * i)),
            pl.BlockSpec((D, H), lambda i: (0, 2 * i + 1)),
        ],
        out_specs=pl.BlockSpec((H, 2 * D), lambda i: (i, 0)),
        out_shape=jax.ShapeDtypeStruct((V // 2, 2 * D), jnp.float32),
    )

    t_out = _TAIL_OUT
    t_blk = (V // 2) // t_out - 1

    def tail_body(tab_ref, in_ref, o_ref):
        del tab_ref
        blk = in_ref[...]
        m = jnp.concatenate([blk[:, :t_out], blk[:, t_out:]], axis=0)
        o_ref[...] = jnp.transpose(m)

    tail = pl.pallas_call(
        tail_body,
        grid=(1,),
        in_specs=[
            pl.BlockSpec(memory_space=pl.ANY),
            pl.BlockSpec((D, 2 * t_out), lambda i: (0, 0)),
        ],
        out_specs=pl.BlockSpec((t_out, 2 * D), lambda i: (t_blk, 0)),
        out_shape=jax.ShapeDtypeStruct((V // 2, 2 * D), jnp.float32),
        input_output_aliases={0: 0},
    )

    def reformat(embt):
        tab = main(embt, embt)
        embt_tail = lax.slice(embt, (0, _MAIN), (D, V))
        return tail(tab, embt_tail)

    return reformat


def _permute_idx(x):
    ch = 2 * _H
    i = x // ch
    q = x - i * ch
    main_pos = i * ch + jnp.where(q < _H, 2 * q, 2 * (q - _H) + 1)
    qt = x - _MAIN
    tail_pos = _MAIN + jnp.where(
        qt < _TAIL_OUT, 2 * qt, 2 * (qt - _TAIL_OUT) + 1)
    return jnp.where(x < _MAIN, main_pos, tail_pos)


@functools.lru_cache(maxsize=None)
def _make_pool(B, L, D):
    LB = L - _LA
    rows_per = B // _NUM_WORKERS
    mesh = plsc.VectorSubcoreMesh(
        core_axis_name="c", subcore_axis_name="s",
        num_cores=_NUM_CORES, num_subcores=_NUM_SUBCORES)

    @functools.partial(
        pl.kernel,
        out_type=jax.ShapeDtypeStruct((B, D), jnp.float32),
        mesh=mesh,
        compiler_params=pltpu.CompilerParams(
            use_tc_tiling_on_sc=False, needs_layout_passes=False),
        scratch_types=[
            pltpu.VMEM((rows_per, L), jnp.int32),
            [pltpu.VMEM((L, D), jnp.float32) for _ in range(_NBUF)],
            [pltpu.SemaphoreType.DMA for _ in range(_NBUF)],
            pltpu.VMEM((rows_per, D), jnp.float32),
        ],
    )
    def pool(x_hbm, tab_hbm, out_hbm, idx_v, bufs, sems, out_v):
        wid = lax.axis_index("s") * _NUM_CORES + lax.axis_index("c")
        base = wid * rows_per
        pltpu.sync_copy(x_hbm.at[pl.ds(base, rows_per)], idx_v)

        unroll = 8
        nvreg = D // _LANES
        inv_l = 1.0 / L

        def fire(r, b):
            pltpu.async_copy(tab_hbm.at[idx_v.at[r, pl.ds(0, _LA)]],
                             bufs[b].at[pl.ds(0, _LA)], sems[b])
            pltpu.async_copy(tab_hbm.at[idx_v.at[r, pl.ds(_LA, LB)]],
                             bufs[b].at[pl.ds(_LA, LB)], sems[b])

        def drain(b):
            pltpu.make_async_copy(tab_hbm.at[pl.ds(0, L)], bufs[b],
                                  sems[b]).wait()

        def accumulate(r, b):
            buf = bufs[b]

            def acc_body(i, accs):
                accs = list(accs)
                for u in range(unroll):
                    row = i * unroll + u
                    for j in range(nvreg):
                        accs[j] = accs[j] + buf[row, pl.ds(j * _LANES, _LANES)]
                return tuple(accs)

            zero = jnp.zeros((_LANES,), jnp.float32)
            accs = lax.fori_loop(0, L // unroll, acc_body, (zero,) * nvreg)
            for j in range(nvreg):
                out_v[r, pl.ds(j * _LANES, _LANES)] = accs[j] * inv_l

        for b in range(_NBUF):
            fire(b, b)

        def ring_body(i, carry):
            r0 = i * _NBUF
            for b in range(_NBUF):
                drain(b)
                accumulate(r0 + b, b)

                @pl.when(r0 + b + _NBUF < rows_per)
                def _():
                    fire(r0 + b + _NBUF, b)
            return carry

        lax.fori_loop(0, rows_per // _NBUF, ring_body, 0)
        pltpu.sync_copy(out_v, out_hbm.at[pl.ds(base, rows_per)])

    return pool


def _mlp(pooled, W1p, b1, W2, b2):
    B, D = pooled.shape
    H = W1p.shape[1]

    def body(p_ref, w1_ref, b1_ref, w2_ref, b2_ref, o_ref):
        h = jnp.dot(p_ref[:], w1_ref[:], preferred_element_type=jnp.float32)
        h = h + b1_ref[:]
        h = jnp.where(h > 0, h, 0.01 * h)
        o = jnp.sum(h * w2_ref[:], axis=1, keepdims=True) + b2_ref[:]
        o_ref[:] = jnp.maximum(o, 0.0)

    out = pl.pallas_call(
        body,
        out_shape=jax.ShapeDtypeStruct((B, 1), jnp.float32),
    )(pooled, W1p, b1.reshape(1, H), W2.reshape(1, H), b2.reshape(1, 1))
    return out[:, 0]


def kernel(x, emb, W1, b1, W2, b2):
    B, L = x.shape
    V, D = emb.shape
    x = _permute_idx(x.astype(jnp.int32))
    tab = _make_reformat(V, D)(emb.T).reshape(V, D)
    pooled = _make_pool(B, L, D)(x, tab)
    return _mlp(pooled, W1, b1, W2, b2)


# speedup vs baseline: 1.0052x; 1.0052x over previous
import functools

import jax
import jax.numpy as jnp
from jax import lax
from jax.experimental import pallas as pl
from jax.experimental.pallas import tpu as pltpu
from jax.experimental.pallas import tpu_sc as plsc

_NUM_CORES = 2
_NUM_SUBCORES = 16
_NUM_WORKERS = _NUM_CORES * _NUM_SUBCORES
_LANES = 16

_LA = 104

_NBUF = 4

_H = 19200
_G = 26
_MAIN = 2 * _H * _G
_TAIL_OUT = 800


@functools.lru_cache(maxsize=None)
def _make_reformat(V, D):
    H = _H
    assert _MAIN + 2 * _TAIL_OUT == V and (V // 2) % _TAIL_OUT == 0

    def body(in0_ref, in1_ref, o_ref):
        m = jnp.concatenate([in0_ref[...], in1_ref[...]], axis=0)
        o_ref[...] = jnp.transpose(m)

    main = pl.pallas_call(
        body,
        grid=(_G,),
        in_specs=[
            pl.BlockSpec((D, H), lambda i: (0, 2 * i)),
            pl.BlockSpec((D, H), lambda i: (0, 2 * i + 1)),
        ],
        out_specs=pl.BlockSpec((H, 2 * D), lambda i: (i, 0)),
        out_shape=jax.ShapeDtypeStruct((V // 2, 2 * D), jnp.float32),
    )

    t_out = _TAIL_OUT
    t_blk = (V // 2) // t_out - 1

    def tail_body(tab_ref, in_ref, o_ref):
        del tab_ref
        blk = in_ref[...]
        m = jnp.concatenate([blk[:, :t_out], blk[:, t_out:]], axis=0)
        o_ref[...] = jnp.transpose(m)

    tail = pl.pallas_call(
        tail_body,
        grid=(1,),
        in_specs=[
            pl.BlockSpec(memory_space=pl.ANY),
            pl.BlockSpec((D, 2 * t_out), lambda i: (0, 0)),
        ],
        out_specs=pl.BlockSpec((t_out, 2 * D), lambda i: (t_blk, 0)),
        out_shape=jax.ShapeDtypeStruct((V // 2, 2 * D), jnp.float32),
        input_output_aliases={0: 0},
    )

    def reformat(embt):
        tab = main(embt, embt)
        embt_tail = lax.slice(embt, (0, _MAIN), (D, V))
        return tail(tab, embt_tail)

    return reformat


def _permute_idx(x):
    ch = 2 * _H
    i = x // ch
    q = x - i * ch
    main_pos = i * ch + jnp.where(q < _H, 2 * q, 2 * (q - _H) + 1)
    qt = x - _MAIN
    tail_pos = _MAIN + jnp.where(
        qt < _TAIL_OUT, 2 * qt, 2 * (qt - _TAIL_OUT) + 1)
    return jnp.where(x < _MAIN, main_pos, tail_pos)


@functools.lru_cache(maxsize=None)
def _make_pool(B, L, D):
    LB = L - _LA
    rows_per = B // _NUM_WORKERS
    mesh = plsc.VectorSubcoreMesh(
        core_axis_name="c", subcore_axis_name="s",
        num_cores=_NUM_CORES, num_subcores=_NUM_SUBCORES)

    @functools.partial(
        pl.kernel,
        out_type=jax.ShapeDtypeStruct((B, D), jnp.float32),
        mesh=mesh,
        compiler_params=pltpu.CompilerParams(
            use_tc_tiling_on_sc=False, needs_layout_passes=False),
        scratch_types=[
            pltpu.VMEM((rows_per, _LA), jnp.int32),
            pltpu.VMEM((rows_per, LB), jnp.int32),
            [pltpu.VMEM((L, D), jnp.float32) for _ in range(_NBUF)],
            [pltpu.SemaphoreType.DMA for _ in range(_NBUF)],
            pltpu.VMEM((rows_per, D), jnp.float32),
        ],
    )
    def pool(xa_hbm, xb_hbm, tab_hbm, out_hbm, idxa_v, idxb_v, bufs, sems,
             out_v):
        wid = lax.axis_index("s") * _NUM_CORES + lax.axis_index("c")
        base = wid * rows_per
        pltpu.sync_copy(xa_hbm.at[pl.ds(base, rows_per)], idxa_v)
        pltpu.sync_copy(xb_hbm.at[pl.ds(base, rows_per)], idxb_v)

        unroll = 8
        nvreg = D // _LANES
        inv_l = 1.0 / L

        def fire(r, b):
            pltpu.async_copy(tab_hbm.at[idxa_v.at[r]],
                             bufs[b].at[pl.ds(0, _LA)], sems[b])
            pltpu.async_copy(tab_hbm.at[idxb_v.at[r]],
                             bufs[b].at[pl.ds(_LA, LB)], sems[b])

        def drain(b):
            pltpu.make_async_copy(tab_hbm.at[pl.ds(0, L)], bufs[b],
                                  sems[b]).wait()

        def accumulate(r, b):
            buf = bufs[b]

            def acc_body(i, accs):
                accs = list(accs)
                for u in range(unroll):
                    row = i * unroll + u
                    for j in range(nvreg):
                        accs[j] = accs[j] + buf[row, pl.ds(j * _LANES, _LANES)]
                return tuple(accs)

            zero = jnp.zeros((_LANES,), jnp.float32)
            accs = lax.fori_loop(0, L // unroll, acc_body, (zero,) * nvreg)
            for j in range(nvreg):
                out_v[r, pl.ds(j * _LANES, _LANES)] = accs[j] * inv_l

        for b in range(_NBUF):
            fire(b, b)

        def ring_body(i, carry):
            r0 = i * _NBUF
            for b in range(_NBUF):
                drain(b)
                accumulate(r0 + b, b)

                @pl.when(r0 + b + _NBUF < rows_per)
                def _():
                    fire(r0 + b + _NBUF, b)
            return carry

        lax.fori_loop(0, rows_per // _NBUF, ring_body, 0)
        pltpu.sync_copy(out_v, out_hbm.at[pl.ds(base, rows_per)])

    return pool


def _mlp(pooled, W1p, b1, W2, b2):
    B, D = pooled.shape
    H = W1p.shape[1]

    def body(p_ref, w1_ref, b1_ref, w2_ref, b2_ref, o_ref):
        h = jnp.dot(p_ref[:], w1_ref[:], preferred_element_type=jnp.float32)
        h = h + b1_ref[:]
        h = jnp.where(h > 0, h, 0.01 * h)
        o = jnp.sum(h * w2_ref[:], axis=1, keepdims=True) + b2_ref[:]
        o_ref[:] = jnp.maximum(o, 0.0)

    out = pl.pallas_call(
        body,
        out_shape=jax.ShapeDtypeStruct((B, 1), jnp.float32),
    )(pooled, W1p, b1.reshape(1, H), W2.reshape(1, H), b2.reshape(1, 1))
    return out[:, 0]


def kernel(x, emb, W1, b1, W2, b2):
    B, L = x.shape
    V, D = emb.shape
    x = _permute_idx(x.astype(jnp.int32))
    xa = x[:, :_LA]
    xb = x[:, _LA:]
    tab = _make_reformat(V, D)(emb.T).reshape(V, D)
    pooled = _make_pool(B, L, D)(xa, xb, tab)
    return _mlp(pooled, W1, b1, W2, b2)


# speedup vs baseline: 1.0070x; 1.0018x over previous
import functools

import jax
import jax.numpy as jnp
from jax import lax
from jax.experimental import pallas as pl
from jax.experimental.pallas import tpu as pltpu
from jax.experimental.pallas import tpu_sc as plsc

_NUM_CORES = 2
_NUM_SUBCORES = 16
_NUM_WORKERS = _NUM_CORES * _NUM_SUBCORES
_LANES = 16

_LA = 104

_NBUF = 4

_H = 19200
_G = 26
_MAIN = 2 * _H * _G
_TAIL_OUT = 800


@functools.lru_cache(maxsize=None)
def _make_reformat(V, D):
    H = _H
    assert _MAIN + 2 * _TAIL_OUT == V and (V // 2) % _TAIL_OUT == 0

    def body(in_ref, o_ref):
        blk = in_ref[...]
        m = jnp.concatenate([blk[:, :H], blk[:, H:]], axis=0)
        o_ref[...] = jnp.transpose(m)

    main = pl.pallas_call(
        body,
        grid=(_G,),
        in_specs=[
            pl.BlockSpec((D, 2 * H), lambda i: (0, i)),
        ],
        out_specs=pl.BlockSpec((H, 2 * D), lambda i: (i, 0)),
        out_shape=jax.ShapeDtypeStruct((V // 2, 2 * D), jnp.float32),
    )

    t_out = _TAIL_OUT
    t_blk = (V // 2) // t_out - 1

    def tail_body(tab_ref, in_ref, o_ref):
        del tab_ref
        blk = in_ref[...]
        m = jnp.concatenate([blk[:, :t_out], blk[:, t_out:]], axis=0)
        o_ref[...] = jnp.transpose(m)

    tail = pl.pallas_call(
        tail_body,
        grid=(1,),
        in_specs=[
            pl.BlockSpec(memory_space=pl.ANY),
            pl.BlockSpec((D, 2 * t_out), lambda i: (0, 0)),
        ],
        out_specs=pl.BlockSpec((t_out, 2 * D), lambda i: (t_blk, 0)),
        out_shape=jax.ShapeDtypeStruct((V // 2, 2 * D), jnp.float32),
        input_output_aliases={0: 0},
    )

    def reformat(embt):
        tab = main(embt)
        embt_tail = lax.slice(embt, (0, _MAIN), (D, V))
        return tail(tab, embt_tail)

    return reformat


def _permute_idx(x):
    ch = 2 * _H
    i = x // ch
    q = x - i * ch
    main_pos = i * ch + jnp.where(q < _H, 2 * q, 2 * (q - _H) + 1)
    qt = x - _MAIN
    tail_pos = _MAIN + jnp.where(
        qt < _TAIL_OUT, 2 * qt, 2 * (qt - _TAIL_OUT) + 1)
    return jnp.where(x < _MAIN, main_pos, tail_pos)


@functools.lru_cache(maxsize=None)
def _make_pool(B, L, D):
    LB = L - _LA
    rows_per = B // _NUM_WORKERS
    mesh = plsc.VectorSubcoreMesh(
        core_axis_name="c", subcore_axis_name="s",
        num_cores=_NUM_CORES, num_subcores=_NUM_SUBCORES)

    @functools.partial(
        pl.kernel,
        out_type=jax.ShapeDtypeStruct((B, D), jnp.float32),
        mesh=mesh,
        compiler_params=pltpu.CompilerParams(
            use_tc_tiling_on_sc=False, needs_layout_passes=False),
        scratch_types=[
            pltpu.VMEM((rows_per, _LA), jnp.int32),
            pltpu.VMEM((rows_per, LB), jnp.int32),
            [pltpu.VMEM((L, D), jnp.float32) for _ in range(_NBUF)],
            [pltpu.SemaphoreType.DMA for _ in range(_NBUF)],
            pltpu.VMEM((rows_per, D), jnp.float32),
        ],
    )
    def pool(xa_hbm, xb_hbm, tab_hbm, out_hbm, idxa_v, idxb_v, bufs, sems,
             out_v):
        wid = lax.axis_index("s") * _NUM_CORES + lax.axis_index("c")
        base = wid * rows_per
        pltpu.sync_copy(xa_hbm.at[pl.ds(base, rows_per)], idxa_v)
        pltpu.sync_copy(xb_hbm.at[pl.ds(base, rows_per)], idxb_v)

        unroll = 8
        nvreg = D // _LANES
        inv_l = 1.0 / L

        def fire(r, b):
            pltpu.async_copy(tab_hbm.at[idxa_v.at[r]],
                             bufs[b].at[pl.ds(0, _LA)], sems[b])
            pltpu.async_copy(tab_hbm.at[idxb_v.at[r]],
                             bufs[b].at[pl.ds(_LA, LB)], sems[b])

        def drain(b):
            pltpu.make_async_copy(tab_hbm.at[pl.ds(0, L)], bufs[b],
                                  sems[b]).wait()

        def accumulate(r, b):
            buf = bufs[b]

            def acc_body(i, accs):
                accs = list(accs)
                for u in range(unroll):
                    row = i * unroll + u
                    for j in range(nvreg):
                        accs[j] = accs[j] + buf[row, pl.ds(j * _LANES, _LANES)]
                return tuple(accs)

            zero = jnp.zeros((_LANES,), jnp.float32)
            accs = lax.fori_loop(0, L // unroll, acc_body, (zero,) * nvreg)
            for j in range(nvreg):
                out_v[r, pl.ds(j * _LANES, _LANES)] = accs[j] * inv_l

        for b in range(_NBUF):
            fire(b, b)

        def ring_body(i, carry):
            r0 = i * _NBUF
            for b in range(_NBUF):
                drain(b)
                accumulate(r0 + b, b)

                @pl.when(r0 + b + _NBUF < rows_per)
                def _():
                    fire(r0 + b + _NBUF, b)
            return carry

        lax.fori_loop(0, rows_per // _NBUF, ring_body, 0)
        pltpu.sync_copy(out_v, out_hbm.at[pl.ds(base, rows_per)])

    return pool


def _mlp(pooled, W1p, b1, W2, b2):
    B, D = pooled.shape
    H = W1p.shape[1]

    def body(p_ref, w1_ref, b1_ref, w2_ref, b2_ref, o_ref):
        h = jnp.dot(p_ref[:], w1_ref[:], preferred_element_type=jnp.float32)
        h = h + b1_ref[:]
        h = jnp.where(h > 0, h, 0.01 * h)
        o = jnp.sum(h * w2_ref[:], axis=1, keepdims=True) + b2_ref[:]
        o_ref[:] = jnp.maximum(o, 0.0)

    out = pl.pallas_call(
        body,
        out_shape=jax.ShapeDtypeStruct((B, 1), jnp.float32),
    )(pooled, W1p, b1.reshape(1, H), W2.reshape(1, H), b2.reshape(1, 1))
    return out[:, 0]


def kernel(x, emb, W1, b1, W2, b2):
    B, L = x.shape
    V, D = emb.shape
    x = _permute_idx(x.astype(jnp.int32))
    xa = x[:, :_LA]
    xb = x[:, _LA:]
    tab = _make_reformat(V, D)(emb.T).reshape(V, D)
    pooled = _make_pool(B, L, D)(xa, xb, tab)
    return _mlp(pooled, W1, b1, W2, b2)
